# baseline (speedup 1.0000x reference)
import functools

import jax
import jax.numpy as jnp
from jax import lax
from jax.experimental import pallas as pl
from jax.experimental.pallas import tpu as pltpu
from jax.experimental.pallas import tpu_sc as plsc

_V = 1000
_E = 16
_B = 1024
_NC = _V * _E
_BB = 32

_NCORE = 2
_NSUB = 16
_NW = _NCORE * _NSUB
_RPW = _B // _NW
_CH = 8

_SC_FIELDS = (3,)
_TC_FIELDS = (0, 1, 2)


def _tc_body(*refs):
    n = len(_TC_FIELDS)
    i_refs, r_refs, o_refs = refs[:n], refs[n:2 * n], refs[2 * n:]
    colv = lax.shift_right_logical(
        lax.broadcasted_iota(jnp.int32, (_BB, _NC), 1), 4
    )
    for i_ref, r_ref, o_ref in zip(i_refs, r_refs, o_refs):
        mask = colv == i_ref[...]
        o_ref[...] = jnp.where(mask, r_ref[1:2, :], r_ref[0:1, :])


def _tc_fill(inputs, tables):
    idx = [inp.reshape(_B, 1) for inp in inputs]
    rows = [jnp.stack([jnp.tile(t[0], _V), jnp.tile(t[1], _V)]) for t in tables]
    n = len(inputs)
    grid = (_B // _BB,)
    in_specs = (
        [pl.BlockSpec((_BB, 1), lambda i: (i, 0)) for _ in range(n)]
        + [pl.BlockSpec((2, _NC), lambda i: (0, 0)) for _ in range(n)]
    )
    out_specs = [pl.BlockSpec((_BB, _NC), lambda i: (i, 0)) for _ in range(n)]
    outs = pl.pallas_call(
        _tc_body,
        grid=grid,
        in_specs=in_specs,
        out_specs=out_specs,
        out_shape=[jax.ShapeDtypeStruct((_B, _NC), jnp.float32)] * n,
    )(*idx, *rows)
    return [o.reshape(_B, _V, _E) for o in outs]


def _sc_body(*refs):
    nf = len(_SC_FIELDS)
    idx_hbms = refs[:nf]
    tmpl_hbms = refs[nf:2 * nf]
    t1_hbms = refs[2 * nf:3 * nf]
    out_hbms = refs[3 * nf:4 * nf]
    buf_v, t1_v, idx_v, sem = refs[4 * nf:]

    wid = lax.axis_index("s") * _NCORE + lax.axis_index("c")
    base_b = wid * _RPW
    lane = lax.iota(jnp.int32, 16)

    for f in range(nf):
        pltpu.sync_copy(tmpl_hbms[f], buf_v)
        pltpu.sync_copy(t1_hbms[f], t1_v)
        pltpu.sync_copy(idx_hbms[f].at[pl.ds(base_b, _RPW)], idx_v)

        copies = []
        for c in range(_RPW // _CH):
            dst = out_hbms[f].at[pl.ds((base_b + c * _CH) * _V, _CH * _V)]
            copies.append(pltpu.async_copy(buf_v, dst, sem))
        for cp in copies:
            cp.wait()

        for h in range(_RPW // 16):
            idx_vals = idx_v[pl.ds(h * 16, 16)]
            vrows = (base_b + h * 16 + lane) * _V + idx_vals
            pltpu.async_copy(t1_v, out_hbms[f].at[vrows], sem).wait()


def _sc_fill(inputs, tables):
    nf = len(inputs)
    mesh = plsc.VectorSubcoreMesh(core_axis_name="c", subcore_axis_name="s")
    run = pl.kernel(
        _sc_body,
        out_type=[jax.ShapeDtypeStruct((_B * _V, _E), jnp.float32)] * nf,
        mesh=mesh,
        compiler_params=pltpu.CompilerParams(use_tc_tiling_on_sc=False),
        scratch_types=[
            pltpu.VMEM((_CH * _V, _E), jnp.float32),
            pltpu.VMEM((16, _E), jnp.float32),
            pltpu.VMEM((_RPW,), jnp.int32),
            pltpu.SemaphoreType.DMA,
        ],
    )
    tmpl = [jnp.broadcast_to(t[0], (_CH * _V, _E)) for t in tables]
    t1 = [jnp.broadcast_to(t[1], (16, _E)) for t in tables]
    outs = run(*inputs, *tmpl, *t1)
    if not isinstance(outs, (list, tuple)):
        outs = [outs]
    return [o.reshape(_B, _V, _E) for o in outs]


def kernel(input0, input1, input2, input3, table0, table1, table2, table3):
    inputs = (input0, input1, input2, input3)
    tables = (table0, table1, table2, table3)

    sc_outs = _sc_fill([inputs[f] for f in _SC_FIELDS],
                       [tables[f] for f in _SC_FIELDS])
    tc_outs = _tc_fill([inputs[f] for f in _TC_FIELDS],
                       [tables[f] for f in _TC_FIELDS])

    out = [None] * 4
    for pos, f in enumerate(_TC_FIELDS):
        out[f] = tc_outs[pos]
    for pos, f in enumerate(_SC_FIELDS):
        out[f] = sc_outs[pos]
    return tuple(out)


# speedup vs baseline: 1.0024x; 1.0024x over previous
import functools

import jax
import jax.numpy as jnp
from jax import lax
from jax.experimental import pallas as pl
from jax.experimental.pallas import tpu as pltpu
from jax.experimental.pallas import tpu_sc as plsc

_V = 1000
_E = 16
_B = 1024
_NC = _V * _E
_BB = 32

_NCORE = 2
_NSUB = 16
_NW = _NCORE * _NSUB
_RPW = _B // _NW
_CH = 8

_SC_FIELDS = (3,)
_TC_FIELDS = (0, 1, 2)


def _tc_body(*refs):
    n = len(_TC_FIELDS)
    i_refs, r_refs, o_refs = refs[:n], refs[n:2 * n], refs[2 * n:]
    colv = lax.shift_right_logical(
        lax.broadcasted_iota(jnp.int32, (_BB, _NC), 1), 4
    )
    for i_ref, r_ref, o_ref in zip(i_refs, r_refs, o_refs):
        mask = colv == i_ref[...]
        o_ref[...] = jnp.where(mask, r_ref[1:2, :], r_ref[0:1, :])


def _tc_fill(inputs, tables):
    idx = [inp.reshape(_B, 1) for inp in inputs]
    rows = [jnp.stack([jnp.tile(t[0], _V), jnp.tile(t[1], _V)]) for t in tables]
    n = len(inputs)
    grid = (_B // _BB,)
    in_specs = (
        [pl.BlockSpec((_BB, 1), lambda i: (i, 0)) for _ in range(n)]
        + [pl.BlockSpec((2, _NC), lambda i: (0, 0)) for _ in range(n)]
    )
    out_specs = [pl.BlockSpec((_BB, _NC), lambda i: (i, 0)) for _ in range(n)]
    outs = pl.pallas_call(
        _tc_body,
        grid=grid,
        in_specs=in_specs,
        out_specs=out_specs,
        out_shape=[jax.ShapeDtypeStruct((_B, _NC), jnp.float32)] * n,
    )(*idx, *rows)
    return [o.reshape(_B, _V, _E) for o in outs]


def _sc_body(*refs):
    nf = len(_SC_FIELDS)
    idx_hbms = refs[:nf]
    tmpl_hbms = refs[nf:2 * nf]
    t1_hbms = refs[2 * nf:3 * nf]
    out_hbms = refs[3 * nf:4 * nf]
    buf_v, t1_v, idx_v, sem = refs[4 * nf:]

    wid = lax.axis_index("s") * _NCORE + lax.axis_index("c")
    base_b = wid * _RPW
    lane = lax.iota(jnp.int32, 16)

    del lane
    for f in range(nf):
        pltpu.sync_copy(tmpl_hbms[f], buf_v)
        pltpu.sync_copy(t1_hbms[f], t1_v)
        pltpu.sync_copy(idx_hbms[f].at[pl.ds(base_b, _RPW)], idx_v)

        copies = []
        for c in range(_RPW // _CH):
            dst = out_hbms[f].at[pl.ds(base_b + c * _CH, _CH)]
            copies.append(pltpu.async_copy(buf_v, dst, sem))
        for cp in copies:
            cp.wait()

        copies = []
        for h in range(_RPW // 16):
            idx_vals = idx_v[pl.ds(h * 16, 16)]
            for j in range(16):
                i = h * 16 + j
                dst = out_hbms[f].at[base_b + i, pl.ds(idx_vals[j], 1)]
                copies.append(pltpu.async_copy(t1_v, dst, sem))
        for cp in copies:
            cp.wait()


def _sc_fill(inputs, tables):
    nf = len(inputs)
    mesh = plsc.VectorSubcoreMesh(core_axis_name="c", subcore_axis_name="s")
    run = pl.kernel(
        _sc_body,
        out_type=[jax.ShapeDtypeStruct((_B, _V, _E), jnp.float32)] * nf,
        mesh=mesh,
        compiler_params=pltpu.CompilerParams(use_tc_tiling_on_sc=False),
        scratch_types=[
            pltpu.VMEM((_CH, _V, _E), jnp.float32),
            pltpu.VMEM((1, _E), jnp.float32),
            pltpu.VMEM((_RPW,), jnp.int32),
            pltpu.SemaphoreType.DMA,
        ],
    )
    tmpl = [jnp.broadcast_to(t[0], (_CH, _V, _E)) for t in tables]
    t1 = [t[1].reshape(1, _E) for t in tables]
    outs = run(*inputs, *tmpl, *t1)
    if not isinstance(outs, (list, tuple)):
        outs = [outs]
    return list(outs)


def kernel(input0, input1, input2, input3, table0, table1, table2, table3):
    inputs = (input0, input1, input2, input3)
    tables = (table0, table1, table2, table3)

    sc_outs = _sc_fill([inputs[f] for f in _SC_FIELDS],
                       [tables[f] for f in _SC_FIELDS])
    tc_outs = _tc_fill([inputs[f] for f in _TC_FIELDS],
                       [tables[f] for f in _TC_FIELDS])

    out = [None] * 4
    for pos, f in enumerate(_TC_FIELDS):
        out[f] = tc_outs[pos]
    for pos, f in enumerate(_SC_FIELDS):
        out[f] = sc_outs[pos]
    return tuple(out)


# speedup vs baseline: 5.6547x; 5.6411x over previous
import jax
import jax.numpy as jnp
from jax import lax
from jax.experimental import pallas as pl

_V = 1000
_E = 16
_B = 1024
_NR = _V * _E
_RB = 800


def _fill_kernel(i0, i1, i2, i3, w0, w1, w2, w3, o0, o1, o2, o3):
    rowv = lax.shift_right_logical(
        pl.program_id(0) * _RB
        + lax.broadcasted_iota(jnp.int32, (_RB, _B), 0),
        4,
    )
    for i_ref, w_ref, o_ref in ((i0, w0, o0), (i1, w1, o1), (i2, w2, o2), (i3, w3, o3)):
        mask = rowv == i_ref[0]
        o_ref[...] = jnp.where(mask, w_ref[:, 1:2], w_ref[:, 0:1])


def kernel(input0, input1, input2, input3, table0, table1, table2, table3):
    inputs = (input0, input1, input2, input3)
    tables = (table0, table1, table2, table3)

    idx = [inp.reshape(1, 1, _B) for inp in inputs]
    cols = [jnp.stack([jnp.tile(t[0], _V), jnp.tile(t[1], _V)], axis=1)
            for t in tables]

    grid = (_NR // _RB,)
    in_specs = (
        [pl.BlockSpec((1, 1, _B), lambda i: (0, 0, 0)) for _ in range(4)]
        + [pl.BlockSpec((_RB, 2), lambda i: (i, 0)) for _ in range(4)]
    )
    out_specs = [pl.BlockSpec((_RB, _B), lambda i: (i, 0)) for _ in range(4)]
    outs = pl.pallas_call(
        _fill_kernel,
        grid=grid,
        in_specs=in_specs,
        out_specs=out_specs,
        out_shape=[jax.ShapeDtypeStruct((_NR, _B), jnp.float32)] * 4,
    )(*idx, *cols)
    return tuple(
        o.reshape(_V, _E, _B).transpose(2, 0, 1) for o in outs
    )


# speedup vs baseline: 5.6796x; 1.0044x over previous
import jax
import jax.numpy as jnp
from jax import lax
from jax.experimental import pallas as pl

_V = 1000
_E = 16
_B = 1024
_NR = _V * _E
_RB = 1000


def _fill_kernel(i0, i1, i2, i3, w0, w1, w2, w3, o0, o1, o2, o3):
    rowv = lax.shift_right_logical(
        pl.program_id(0) * _RB
        + lax.broadcasted_iota(jnp.int32, (_RB, _B), 0),
        4,
    )
    for i_ref, w_ref, o_ref in ((i0, w0, o0), (i1, w1, o1), (i2, w2, o2), (i3, w3, o3)):
        mask = rowv == i_ref[0]
        o_ref[...] = jnp.where(mask, w_ref[:, 1:2], w_ref[:, 0:1])


def kernel(input0, input1, input2, input3, table0, table1, table2, table3):
    inputs = (input0, input1, input2, input3)
    tables = (table0, table1, table2, table3)

    idx = [inp.reshape(1, 1, _B) for inp in inputs]
    cols = [jnp.stack([jnp.tile(t[0], _V), jnp.tile(t[1], _V)], axis=1)
            for t in tables]

    grid = (_NR // _RB,)
    in_specs = (
        [pl.BlockSpec((1, 1, _B), lambda i: (0, 0, 0)) for _ in range(4)]
        + [pl.BlockSpec((_RB, 2), lambda i: (i, 0)) for _ in range(4)]
    )
    out_specs = [pl.BlockSpec((_RB, _B), lambda i: (i, 0)) for _ in range(4)]
    outs = pl.pallas_call(
        _fill_kernel,
        grid=grid,
        in_specs=in_specs,
        out_specs=out_specs,
        out_shape=[jax.ShapeDtypeStruct((_NR, _B), jnp.float32)] * 4,
    )(*idx, *cols)
    return tuple(
        o.reshape(_V, _E, _B).transpose(2, 0, 1) for o in outs
    )
